# baseline (speedup 1.0000x reference)
import functools

import jax
import jax.numpy as jnp
from jax import lax
from jax.experimental import pallas as pl
from jax.experimental.pallas import tpu as pltpu
from jax.experimental.pallas import tpu_sc as plsc

N = 10000
E = 320000
D = 128

NC = 2
NS = 16
NW = NC * NS

EP = E // NW
C = 96
NCHUNK = 105
EPP = NCHUNK * C
CP = 80

HPAD = 10240
HP = HPAD // NS
NPAD = 10240
RP = NPAD // NS
RSTG = 128

_mesh = plsc.VectorSubcoreMesh(core_axis_name="c", subcore_axis_name="s")


def _zero_fill_1d(ref, n):
    z = jnp.zeros((16,), jnp.float32)

    def body(i, _):
        ref[pl.ds(i * 16, 16)] = z
        return 0

    lax.fori_loop(0, n // 16, body, 0)


@functools.partial(
    pl.kernel,
    mesh=_mesh,
    out_type=jax.ShapeDtypeStruct((NC * HPAD,), jnp.float32),
    scratch_types=[
        pltpu.VMEM_SHARED((HPAD,), jnp.float32),
        pltpu.VMEM((NCHUNK, C), jnp.int32),
        pltpu.VMEM((C,), jnp.float32),
        pltpu.VMEM((HP,), jnp.float32),
        pltpu.SemaphoreType.DMA,
        pltpu.SemaphoreType.DMA,
    ],
)
def _deg_kernel(dst_hbm, out_hbm, hist_sp, didx_v, ones_v, stage_v,
                sem0, sem1):
    cid = lax.axis_index("c")
    sid = lax.axis_index("s")
    wid = cid * NS + sid

    pltpu.sync_copy(dst_hbm.at[wid], didx_v)
    _zero_fill_1d(stage_v, HP)
    one = jnp.ones((16,), jnp.float32)

    def fill_ones(i, _):
        ones_v[pl.ds(i * 16, 16)] = one
        return 0

    lax.fori_loop(0, C // 16, fill_ones, 0)

    pltpu.sync_copy(stage_v, hist_sp.at[pl.ds(sid * HP, HP)])
    plsc.subcore_barrier()

    sem = (sem0, sem1)

    def add_ones(k, s):
        pltpu.async_copy(ones_v, hist_sp.at[didx_v.at[k]], sem[s], add=True)

    def add_wait(s):
        pltpu.make_async_copy(ones_v, hist_sp.at[didx_v.at[0]], sem[s]).wait()

    def chunk_pair(i, _):
        @pl.when(i >= 1)
        def _():
            add_wait(0)
        add_ones(2 * i, 0)

        @pl.when(i >= 1)
        def _():
            add_wait(1)
        add_ones(2 * i + 1, 1)
        return 0

    lax.fori_loop(0, (NCHUNK - 1) // 2, chunk_pair, 0)
    add_wait(0)
    add_ones(NCHUNK - 1, 0)
    add_wait(0)
    add_wait(1)
    plsc.subcore_barrier()

    pltpu.sync_copy(hist_sp.at[pl.ds(sid * HP, HP)], stage_v)
    pltpu.sync_copy(stage_v, out_hbm.at[pl.ds(cid * HPAD + sid * HP, HP)])


@functools.partial(
    pl.kernel,
    mesh=_mesh,
    out_type=jax.ShapeDtypeStruct((NC * NPAD, D), jnp.float32),
    scratch_types=[
        pltpu.VMEM_SHARED((NPAD, D), jnp.float32),
        pltpu.VMEM((C, D), jnp.float32),
        pltpu.VMEM((C, D), jnp.float32),
        pltpu.VMEM((EPP,), jnp.int32),
        pltpu.VMEM((NCHUNK, C), jnp.int32),
        pltpu.SemaphoreType.DMA,
        pltpu.SemaphoreType.DMA,
        pltpu.SemaphoreType.DMA,
        pltpu.SemaphoreType.DMA,
    ],
)
def _scatter_kernel(y_hbm, src_hbm, dst_hbm, out_hbm,
                    accum_sp, rows0_v, rows1_v, sidx_v, didx_v,
                    gsem0, gsem1, ssem0, ssem1):
    cid = lax.axis_index("c")
    sid = lax.axis_index("s")
    wid = cid * NS + sid

    pltpu.sync_copy(src_hbm.at[wid], sidx_v)
    pltpu.sync_copy(dst_hbm.at[wid], didx_v)

    z = jnp.zeros((16,), jnp.float32)

    def zrow(i, _):
        rows0_v[i // (D // 16), pl.ds((i % (D // 16)) * 16, 16)] = z
        return 0

    lax.fori_loop(0, C * (D // 16), zrow, 0)
    for j in range(RP // CP):
        pltpu.sync_copy(rows0_v.at[pl.ds(0, CP)],
                        accum_sp.at[pl.ds(sid * RP + j * CP, CP)])
    plsc.subcore_barrier()

    rows = (rows0_v, rows1_v)
    gsem = (gsem0, gsem1)
    ssem = (ssem0, ssem1)

    def gather(k, buf):
        pltpu.async_copy(y_hbm.at[sidx_v.at[pl.ds(k * C, C)]], rows[buf],
                         gsem[buf])

    def gather_wait(k, buf):
        pltpu.make_async_copy(y_hbm.at[sidx_v.at[pl.ds(k * C, C)]], rows[buf],
                              gsem[buf]).wait()

    def scatter(k, buf):
        pltpu.sync_copy(rows[buf], accum_sp.at[didx_v.at[k]], add=True)

    del ssem

    gather(0, 0)
    gather(1, 1)

    def pair(i, _):
        k0 = 2 * i
        k1 = k0 + 1
        gather_wait(k0, 0)
        scatter(k0, 0)
        gather(k0 + 2, 0)
        gather_wait(k1, 1)
        scatter(k1, 1)

        @pl.when(k1 + 2 <= NCHUNK - 1)
        def _():
            gather(k1 + 2, 1)
        return 0

    lax.fori_loop(0, (NCHUNK - 1) // 2, pair, 0)
    gather_wait(NCHUNK - 1, 0)
    scatter(NCHUNK - 1, 0)
    plsc.subcore_barrier()

    for j in range(RP // CP):
        r = sid * RP + j * CP
        pltpu.sync_copy(accum_sp.at[pl.ds(r, CP)], rows0_v.at[pl.ds(0, CP)])
        pltpu.sync_copy(rows0_v.at[pl.ds(0, CP)],
                        out_hbm.at[pl.ds(cid * NPAD + r, CP)])


BN = 400
GRID = N // BN


def _layer1_body(x_ref, w_ref, p0_ref, p1_ref, y_ref, dinv_ref):
    deg = p0_ref[...] + p1_ref[...] + 1.0
    dinv = lax.rsqrt(deg)
    z = x_ref[...] * dinv
    y_ref[...] = jnp.dot(z, w_ref[...], preferred_element_type=jnp.float32)
    dinv_ref[...] = dinv


def _layer1_tc(x, W1, p0, p1):
    return pl.pallas_call(
        _layer1_body,
        grid=(GRID,),
        in_specs=[
            pl.BlockSpec((BN, D), lambda i: (i, 0)),
            pl.BlockSpec((D, D), lambda i: (0, 0)),
            pl.BlockSpec((BN, 1), lambda i: (i, 0)),
            pl.BlockSpec((BN, 1), lambda i: (i, 0)),
        ],
        out_specs=[
            pl.BlockSpec((BN, D), lambda i: (i, 0)),
            pl.BlockSpec((BN, 1), lambda i: (i, 0)),
        ],
        out_shape=[
            jax.ShapeDtypeStruct((N, D), jnp.float32),
            jax.ShapeDtypeStruct((N, 1), jnp.float32),
        ],
    )(x, W1, p0, p1)


def _mid_body(s0_ref, s1_ref, y1_ref, dinv_ref, w_ref, b_ref, y2_ref):
    dinv = dinv_ref[...]
    t = dinv * (s0_ref[...] + s1_ref[...] + y1_ref[...]) + b_ref[...]
    h = jnp.maximum(t, 0.0)
    y2_ref[...] = dinv * jnp.dot(h, w_ref[...],
                                 preferred_element_type=jnp.float32)


def _mid_tc(s0, s1, y1, dinv, W2, b1):
    return pl.pallas_call(
        _mid_body,
        grid=(GRID,),
        in_specs=[
            pl.BlockSpec((BN, D), lambda i: (i, 0)),
            pl.BlockSpec((BN, D), lambda i: (i, 0)),
            pl.BlockSpec((BN, D), lambda i: (i, 0)),
            pl.BlockSpec((BN, 1), lambda i: (i, 0)),
            pl.BlockSpec((D, D), lambda i: (0, 0)),
            pl.BlockSpec((1, D), lambda i: (0, 0)),
        ],
        out_specs=pl.BlockSpec((BN, D), lambda i: (i, 0)),
        out_shape=jax.ShapeDtypeStruct((N, D), jnp.float32),
    )(s0, s1, y1, dinv, W2, b1)


def _final_body(s0_ref, s1_ref, y2_ref, dinv_ref, b_ref, o_ref):
    o_ref[...] = (dinv_ref[...] * (s0_ref[...] + s1_ref[...] + y2_ref[...])
                  + b_ref[...])


def _final_tc(s0, s1, y2, dinv, b2):
    return pl.pallas_call(
        _final_body,
        grid=(GRID,),
        in_specs=[
            pl.BlockSpec((BN, D), lambda i: (i, 0)),
            pl.BlockSpec((BN, D), lambda i: (i, 0)),
            pl.BlockSpec((BN, D), lambda i: (i, 0)),
            pl.BlockSpec((BN, 1), lambda i: (i, 0)),
            pl.BlockSpec((1, D), lambda i: (0, 0)),
        ],
        out_specs=pl.BlockSpec((BN, D), lambda i: (i, 0)),
        out_shape=jax.ShapeDtypeStruct((N, D), jnp.float32),
    )(s0, s1, y2, dinv, b2)


def kernel(x, edge_index, W1, b1, W2, b2):
    src = edge_index[0]
    dst = edge_index[1]
    src2 = jnp.pad(src.reshape(NW, EP), ((0, 0), (0, EPP - EP)))
    dst3 = jnp.pad(dst.reshape(NW, EP), ((0, 0), (0, EPP - EP)),
                   constant_values=NPAD - 1).reshape(NW, NCHUNK, C)

    hist = _deg_kernel(dst3)
    p0 = hist[:N].reshape(N, 1)
    p1 = hist[HPAD:HPAD + N].reshape(N, 1)
    y1, dinv = _layer1_tc(x, W1, p0, p1)

    s1 = _scatter_kernel(y1, src2, dst3)
    y2 = _mid_tc(s1[:N], s1[NPAD:NPAD + N], y1, dinv, W2, b1.reshape(1, D))

    s2 = _scatter_kernel(y2, src2, dst3)
    out = _final_tc(s2[:N], s2[NPAD:NPAD + N], y2, dinv, b2.reshape(1, D))
    return out


# speedup vs baseline: 1.4771x; 1.4771x over previous
import functools

import jax
import jax.numpy as jnp
from jax import lax
from jax.experimental import pallas as pl
from jax.experimental.pallas import tpu as pltpu
from jax.experimental.pallas import tpu_sc as plsc

N = 10000
E = 320000
D = 128

NC = 2
NS = 16
NW = NC * NS

EP = E // NW
C = 80
NCHUNK = EP // C
EPP = NCHUNK * C
CP = 80

HPAD = 10240
HP = HPAD // NS
NPAD = 10240
RP = NPAD // NS
RSTG = 128

_mesh = plsc.VectorSubcoreMesh(core_axis_name="c", subcore_axis_name="s")


def _zero_fill_1d(ref, n):
    z = jnp.zeros((16,), jnp.float32)

    def body(i, _):
        ref[pl.ds(i * 16, 16)] = z
        return 0

    lax.fori_loop(0, n // 16, body, 0)


@functools.partial(
    pl.kernel,
    mesh=_mesh,
    out_type=jax.ShapeDtypeStruct((NC * HPAD,), jnp.float32),
    scratch_types=[
        pltpu.VMEM_SHARED((HPAD,), jnp.float32),
        pltpu.VMEM((NCHUNK, C), jnp.int32),
        pltpu.VMEM((C,), jnp.float32),
        pltpu.VMEM((HP,), jnp.float32),
        pltpu.SemaphoreType.DMA,
        pltpu.SemaphoreType.DMA,
    ],
)
def _deg_kernel(dst_hbm, out_hbm, hist_sp, didx_v, ones_v, stage_v,
                sem0, sem1):
    cid = lax.axis_index("c")
    sid = lax.axis_index("s")
    wid = cid * NS + sid

    pltpu.sync_copy(dst_hbm.at[wid], didx_v)
    _zero_fill_1d(stage_v, HP)
    one = jnp.ones((16,), jnp.float32)

    def fill_ones(i, _):
        ones_v[pl.ds(i * 16, 16)] = one
        return 0

    lax.fori_loop(0, C // 16, fill_ones, 0)

    pltpu.sync_copy(stage_v, hist_sp.at[pl.ds(sid * HP, HP)])
    plsc.subcore_barrier()

    sem = (sem0, sem1)

    def add_ones(k, s):
        pltpu.async_copy(ones_v, hist_sp.at[didx_v.at[k]], sem[s], add=True)

    def add_wait(s):
        pltpu.make_async_copy(ones_v, hist_sp.at[didx_v.at[0]], sem[s]).wait()

    def chunk_pair(i, _):
        @pl.when(i >= 1)
        def _():
            add_wait(0)
        add_ones(2 * i, 0)

        @pl.when(i >= 1)
        def _():
            add_wait(1)
        add_ones(2 * i + 1, 1)
        return 0

    lax.fori_loop(0, (NCHUNK - 1) // 2, chunk_pair, 0)
    add_wait(0)
    add_ones(NCHUNK - 1, 0)
    add_wait(0)
    add_wait(1)
    plsc.subcore_barrier()

    pltpu.sync_copy(hist_sp.at[pl.ds(sid * HP, HP)], stage_v)
    pltpu.sync_copy(stage_v, out_hbm.at[pl.ds(cid * HPAD + sid * HP, HP)])


@functools.partial(
    pl.kernel,
    mesh=_mesh,
    out_type=jax.ShapeDtypeStruct((NC * NPAD, D), jnp.float32),
    scratch_types=[
        pltpu.VMEM_SHARED((NPAD, D), jnp.float32),
        pltpu.VMEM((C, D), jnp.float32),
        pltpu.VMEM((C, D), jnp.float32),
        pltpu.VMEM((EPP,), jnp.int32),
        pltpu.VMEM((NCHUNK, C), jnp.int32),
        pltpu.SemaphoreType.DMA,
        pltpu.SemaphoreType.DMA,
        pltpu.SemaphoreType.DMA,
        pltpu.SemaphoreType.DMA,
    ],
)
def _scatter_kernel(y_hbm, src_hbm, dst_hbm, out_hbm,
                    accum_sp, rows0_v, rows1_v, sidx_v, didx_v,
                    gsem0, gsem1, ssem0, ssem1):
    cid = lax.axis_index("c")
    sid = lax.axis_index("s")
    wid = cid * NS + sid

    pltpu.sync_copy(src_hbm.at[wid], sidx_v)
    pltpu.sync_copy(dst_hbm.at[wid], didx_v)

    z = jnp.zeros((16,), jnp.float32)

    def zrow(i, _):
        rows0_v[i // (D // 16), pl.ds((i % (D // 16)) * 16, 16)] = z
        return 0

    lax.fori_loop(0, C * (D // 16), zrow, 0)
    for j in range(RP // CP):
        pltpu.sync_copy(rows0_v.at[pl.ds(0, CP)],
                        accum_sp.at[pl.ds(sid * RP + j * CP, CP)])
    plsc.subcore_barrier()

    rows = (rows0_v, rows1_v)
    gsem = (gsem0, gsem1)
    ssem = (ssem0, ssem1)

    def gather(k, buf):
        pltpu.async_copy(y_hbm.at[sidx_v.at[pl.ds(k * C, C)]], rows[buf],
                         gsem[buf])

    def gather_wait(k, buf):
        pltpu.make_async_copy(y_hbm.at[sidx_v.at[pl.ds(k * C, C)]], rows[buf],
                              gsem[buf]).wait()

    def scatter(k, buf):
        pltpu.sync_copy(rows[buf], accum_sp.at[didx_v.at[k]], add=True)

    del ssem

    gather(0, 0)
    gather(1, 1)

    def pair(i, _):
        k0 = 2 * i
        k1 = k0 + 1
        gather_wait(k0, 0)
        scatter(k0, 0)
        gather(k0 + 2, 0)
        gather_wait(k1, 1)
        scatter(k1, 1)

        @pl.when(k1 + 2 <= NCHUNK - 1)
        def _():
            gather(k1 + 2, 1)
        return 0

    lax.fori_loop(0, (NCHUNK - 1) // 2, pair, 0)
    gather_wait(NCHUNK - 1, 0)
    scatter(NCHUNK - 1, 0)
    plsc.subcore_barrier()

    for j in range(RP // CP):
        r = sid * RP + j * CP
        pltpu.sync_copy(accum_sp.at[pl.ds(r, CP)], rows0_v.at[pl.ds(0, CP)])
        pltpu.sync_copy(rows0_v.at[pl.ds(0, CP)],
                        out_hbm.at[pl.ds(cid * NPAD + r, CP)])


BN = 400
GRID = N // BN


def _layer1_body(x_ref, w_ref, p0_ref, p1_ref, y_ref, dinv_ref):
    deg = p0_ref[...] + p1_ref[...] + 1.0
    dinv = lax.rsqrt(deg)
    z = x_ref[...] * dinv
    y_ref[...] = jnp.dot(z, w_ref[...], preferred_element_type=jnp.float32)
    dinv_ref[...] = dinv


def _layer1_tc(x, W1, p0, p1):
    return pl.pallas_call(
        _layer1_body,
        grid=(GRID,),
        in_specs=[
            pl.BlockSpec((BN, D), lambda i: (i, 0)),
            pl.BlockSpec((D, D), lambda i: (0, 0)),
            pl.BlockSpec((BN, 1), lambda i: (i, 0)),
            pl.BlockSpec((BN, 1), lambda i: (i, 0)),
        ],
        out_specs=[
            pl.BlockSpec((BN, D), lambda i: (i, 0)),
            pl.BlockSpec((BN, 1), lambda i: (i, 0)),
        ],
        out_shape=[
            jax.ShapeDtypeStruct((N, D), jnp.float32),
            jax.ShapeDtypeStruct((N, 1), jnp.float32),
        ],
    )(x, W1, p0, p1)


def _mid_body(s0_ref, s1_ref, y1_ref, dinv_ref, w_ref, b_ref, y2_ref):
    dinv = dinv_ref[...]
    t = dinv * (s0_ref[...] + s1_ref[...] + y1_ref[...]) + b_ref[...]
    h = jnp.maximum(t, 0.0)
    y2_ref[...] = dinv * jnp.dot(h, w_ref[...],
                                 preferred_element_type=jnp.float32)


def _mid_tc(s0, s1, y1, dinv, W2, b1):
    return pl.pallas_call(
        _mid_body,
        grid=(GRID,),
        in_specs=[
            pl.BlockSpec((BN, D), lambda i: (i, 0)),
            pl.BlockSpec((BN, D), lambda i: (i, 0)),
            pl.BlockSpec((BN, D), lambda i: (i, 0)),
            pl.BlockSpec((BN, 1), lambda i: (i, 0)),
            pl.BlockSpec((D, D), lambda i: (0, 0)),
            pl.BlockSpec((1, D), lambda i: (0, 0)),
        ],
        out_specs=pl.BlockSpec((BN, D), lambda i: (i, 0)),
        out_shape=jax.ShapeDtypeStruct((N, D), jnp.float32),
    )(s0, s1, y1, dinv, W2, b1)


def _final_body(s0_ref, s1_ref, y2_ref, dinv_ref, b_ref, o_ref):
    o_ref[...] = (dinv_ref[...] * (s0_ref[...] + s1_ref[...] + y2_ref[...])
                  + b_ref[...])


def _final_tc(s0, s1, y2, dinv, b2):
    return pl.pallas_call(
        _final_body,
        grid=(GRID,),
        in_specs=[
            pl.BlockSpec((BN, D), lambda i: (i, 0)),
            pl.BlockSpec((BN, D), lambda i: (i, 0)),
            pl.BlockSpec((BN, D), lambda i: (i, 0)),
            pl.BlockSpec((BN, 1), lambda i: (i, 0)),
            pl.BlockSpec((1, D), lambda i: (0, 0)),
        ],
        out_specs=pl.BlockSpec((BN, D), lambda i: (i, 0)),
        out_shape=jax.ShapeDtypeStruct((N, D), jnp.float32),
    )(s0, s1, y2, dinv, b2)


def kernel(x, edge_index, W1, b1, W2, b2):
    src = edge_index[0]
    dst = edge_index[1]
    src2 = src.reshape(NW, EP)
    dst3 = dst.reshape(NW, NCHUNK, C)

    hist = _deg_kernel(dst3)
    p0 = hist[:N].reshape(N, 1)
    p1 = hist[HPAD:HPAD + N].reshape(N, 1)
    y1, dinv = _layer1_tc(x, W1, p0, p1)

    s1 = _scatter_kernel(y1, src2, dst3)
    y2 = _mid_tc(s1[:N], s1[NPAD:NPAD + N], y1, dinv, W2, b1.reshape(1, D))

    s2 = _scatter_kernel(y2, src2, dst3)
    out = _final_tc(s2[:N], s2[NPAD:NPAD + N], y2, dinv, b2.reshape(1, D))
    return out


# speedup vs baseline: 1.4823x; 1.0035x over previous
import functools

import jax
import jax.numpy as jnp
from jax import lax
from jax.experimental import pallas as pl
from jax.experimental.pallas import tpu as pltpu
from jax.experimental.pallas import tpu_sc as plsc

N = 10000
E = 320000
D = 128

NC = 2
NS = 16
NW = NC * NS

EP = E // NW
C = 80
NCHUNK = EP // C
EPP = NCHUNK * C
CP = 80

HPAD = 10240
HP = HPAD // NS
NPAD = 10240
RP = NPAD // NS
RSTG = 128

_mesh = plsc.VectorSubcoreMesh(core_axis_name="c", subcore_axis_name="s")


def _zero_fill_1d(ref, n):
    z = jnp.zeros((16,), jnp.float32)

    def body(i, _):
        ref[pl.ds(i * 16, 16)] = z
        return 0

    lax.fori_loop(0, n // 16, body, 0)


@functools.partial(
    pl.kernel,
    mesh=_mesh,
    out_type=jax.ShapeDtypeStruct((NC * HPAD,), jnp.float32),
    scratch_types=[
        pltpu.VMEM_SHARED((HPAD,), jnp.float32),
        pltpu.VMEM((NCHUNK, C), jnp.int32),
        pltpu.VMEM((C,), jnp.float32),
        pltpu.VMEM((HP,), jnp.float32),
        pltpu.SemaphoreType.DMA,
        pltpu.SemaphoreType.DMA,
    ],
)
def _deg_kernel(dst_hbm, out_hbm, hist_sp, didx_v, ones_v, stage_v,
                sem0, sem1):
    cid = lax.axis_index("c")
    sid = lax.axis_index("s")
    wid = cid * NS + sid

    pltpu.sync_copy(dst_hbm.at[wid], didx_v)
    _zero_fill_1d(stage_v, HP)
    one = jnp.ones((16,), jnp.float32)

    def fill_ones(i, _):
        ones_v[pl.ds(i * 16, 16)] = one
        return 0

    lax.fori_loop(0, C // 16, fill_ones, 0)

    pltpu.sync_copy(stage_v, hist_sp.at[pl.ds(sid * HP, HP)])
    plsc.subcore_barrier()

    sem = (sem0, sem1)

    def add_ones(k, s):
        pltpu.async_copy(ones_v, hist_sp.at[didx_v.at[k]], sem[s], add=True)

    def add_wait(s):
        pltpu.make_async_copy(ones_v, hist_sp.at[didx_v.at[0]], sem[s]).wait()

    def chunk_pair(i, _):
        @pl.when(i >= 1)
        def _():
            add_wait(0)
        add_ones(2 * i, 0)

        @pl.when(i >= 1)
        def _():
            add_wait(1)
        add_ones(2 * i + 1, 1)
        return 0

    lax.fori_loop(0, (NCHUNK - 1) // 2, chunk_pair, 0)
    add_wait(0)
    add_ones(NCHUNK - 1, 0)
    add_wait(0)
    add_wait(1)
    plsc.subcore_barrier()

    pltpu.sync_copy(hist_sp.at[pl.ds(sid * HP, HP)], stage_v)
    pltpu.sync_copy(stage_v, out_hbm.at[pl.ds(cid * HPAD + sid * HP, HP)])


@functools.partial(
    pl.kernel,
    mesh=_mesh,
    out_type=jax.ShapeDtypeStruct((NC * NPAD, D), jnp.float32),
    scratch_types=[
        pltpu.VMEM_SHARED((NPAD, D), jnp.float32),
        pltpu.VMEM((C, D), jnp.float32),
        pltpu.VMEM((C, D), jnp.float32),
        pltpu.VMEM((EPP,), jnp.int32),
        pltpu.VMEM((NCHUNK, C), jnp.int32),
        pltpu.SemaphoreType.DMA,
        pltpu.SemaphoreType.DMA,
        pltpu.SemaphoreType.DMA,
        pltpu.SemaphoreType.DMA,
    ],
)
def _scatter_kernel(y_hbm, src_hbm, dst_hbm, out_hbm,
                    accum_sp, rows0_v, rows1_v, sidx_v, didx_v,
                    gsem0, gsem1, ssem0, ssem1):
    cid = lax.axis_index("c")
    sid = lax.axis_index("s")
    wid = cid * NS + sid

    pltpu.sync_copy(src_hbm.at[wid], sidx_v)
    pltpu.sync_copy(dst_hbm.at[wid], didx_v)

    z = jnp.zeros((16,), jnp.float32)

    def zrow(i, _):
        rows0_v[i // (D // 16), pl.ds((i % (D // 16)) * 16, 16)] = z
        return 0

    lax.fori_loop(0, C * (D // 16), zrow, 0)
    for j in range(RP // CP):
        pltpu.sync_copy(rows0_v.at[pl.ds(0, CP)],
                        accum_sp.at[pl.ds(sid * RP + j * CP, CP)])
    plsc.subcore_barrier()

    rows = (rows0_v, rows1_v)
    gsem = (gsem0, gsem1)
    ssem = (ssem0, ssem1)

    def gather(k, buf):
        pltpu.async_copy(y_hbm.at[sidx_v.at[pl.ds(k * C, C)]], rows[buf],
                         gsem[buf])

    def gather_wait(k, buf):
        pltpu.make_async_copy(y_hbm.at[sidx_v.at[pl.ds(k * C, C)]], rows[buf],
                              gsem[buf]).wait()

    def scatter(k, buf):
        pltpu.sync_copy(rows[buf], accum_sp.at[didx_v.at[k]], add=True)

    del ssem

    gather(0, 0)
    gather(1, 1)

    def pair(i, _):
        k0 = 2 * i
        k1 = k0 + 1
        gather_wait(k0, 0)
        scatter(k0, 0)
        gather(k0 + 2, 0)
        gather_wait(k1, 1)
        scatter(k1, 1)

        @pl.when(k1 + 2 <= NCHUNK - 1)
        def _():
            gather(k1 + 2, 1)
        return 0

    lax.fori_loop(0, (NCHUNK - 1) // 2, pair, 0)
    gather_wait(NCHUNK - 1, 0)
    scatter(NCHUNK - 1, 0)
    plsc.subcore_barrier()

    r = sid * RP
    pltpu.sync_copy(accum_sp.at[pl.ds(r, RP)],
                    out_hbm.at[pl.ds(cid * NPAD + r, RP)])


BN = 400
GRID = N // BN


def _layer1_body(x_ref, w_ref, p0_ref, p1_ref, y_ref, dinv_ref):
    deg = p0_ref[...] + p1_ref[...] + 1.0
    dinv = lax.rsqrt(deg)
    z = x_ref[...] * dinv
    y_ref[...] = jnp.dot(z, w_ref[...], preferred_element_type=jnp.float32)
    dinv_ref[...] = dinv


def _layer1_tc(x, W1, p0, p1):
    return pl.pallas_call(
        _layer1_body,
        grid=(GRID,),
        in_specs=[
            pl.BlockSpec((BN, D), lambda i: (i, 0)),
            pl.BlockSpec((D, D), lambda i: (0, 0)),
            pl.BlockSpec((BN, 1), lambda i: (i, 0)),
            pl.BlockSpec((BN, 1), lambda i: (i, 0)),
        ],
        out_specs=[
            pl.BlockSpec((BN, D), lambda i: (i, 0)),
            pl.BlockSpec((BN, 1), lambda i: (i, 0)),
        ],
        out_shape=[
            jax.ShapeDtypeStruct((N, D), jnp.float32),
            jax.ShapeDtypeStruct((N, 1), jnp.float32),
        ],
    )(x, W1, p0, p1)


def _mid_body(s0_ref, s1_ref, y1_ref, dinv_ref, w_ref, b_ref, y2_ref):
    dinv = dinv_ref[...]
    t = dinv * (s0_ref[...] + s1_ref[...] + y1_ref[...]) + b_ref[...]
    h = jnp.maximum(t, 0.0)
    y2_ref[...] = dinv * jnp.dot(h, w_ref[...],
                                 preferred_element_type=jnp.float32)


def _mid_tc(s0, s1, y1, dinv, W2, b1):
    return pl.pallas_call(
        _mid_body,
        grid=(GRID,),
        in_specs=[
            pl.BlockSpec((BN, D), lambda i: (i, 0)),
            pl.BlockSpec((BN, D), lambda i: (i, 0)),
            pl.BlockSpec((BN, D), lambda i: (i, 0)),
            pl.BlockSpec((BN, 1), lambda i: (i, 0)),
            pl.BlockSpec((D, D), lambda i: (0, 0)),
            pl.BlockSpec((1, D), lambda i: (0, 0)),
        ],
        out_specs=pl.BlockSpec((BN, D), lambda i: (i, 0)),
        out_shape=jax.ShapeDtypeStruct((N, D), jnp.float32),
    )(s0, s1, y1, dinv, W2, b1)


def _final_body(s0_ref, s1_ref, y2_ref, dinv_ref, b_ref, o_ref):
    o_ref[...] = (dinv_ref[...] * (s0_ref[...] + s1_ref[...] + y2_ref[...])
                  + b_ref[...])


def _final_tc(s0, s1, y2, dinv, b2):
    return pl.pallas_call(
        _final_body,
        grid=(GRID,),
        in_specs=[
            pl.BlockSpec((BN, D), lambda i: (i, 0)),
            pl.BlockSpec((BN, D), lambda i: (i, 0)),
            pl.BlockSpec((BN, D), lambda i: (i, 0)),
            pl.BlockSpec((BN, 1), lambda i: (i, 0)),
            pl.BlockSpec((1, D), lambda i: (0, 0)),
        ],
        out_specs=pl.BlockSpec((BN, D), lambda i: (i, 0)),
        out_shape=jax.ShapeDtypeStruct((N, D), jnp.float32),
    )(s0, s1, y2, dinv, b2)


def kernel(x, edge_index, W1, b1, W2, b2):
    src = edge_index[0]
    dst = edge_index[1]
    src2 = src.reshape(NW, EP)
    dst3 = dst.reshape(NW, NCHUNK, C)

    hist = _deg_kernel(dst3)
    p0 = hist[:N].reshape(N, 1)
    p1 = hist[HPAD:HPAD + N].reshape(N, 1)
    y1, dinv = _layer1_tc(x, W1, p0, p1)

    s1 = _scatter_kernel(y1, src2, dst3)
    y2 = _mid_tc(s1[:N], s1[NPAD:NPAD + N], y1, dinv, W2, b1.reshape(1, D))

    s2 = _scatter_kernel(y2, src2, dst3)
    out = _final_tc(s2[:N], s2[NPAD:NPAD + N], y2, dinv, b2.reshape(1, D))
    return out


# speedup vs baseline: 1.5606x; 1.0528x over previous
import functools

import jax
import jax.numpy as jnp
from jax import lax
from jax.experimental import pallas as pl
from jax.experimental.pallas import tpu as pltpu
from jax.experimental.pallas import tpu_sc as plsc

N = 10000
E = 320000
D = 128

NC = 2
NS = 16
NW = NC * NS

EP = E // NW
C = 80
NCHUNK = EP // C
EPP = NCHUNK * C
CP = 80

HPAD = 10240
HP = HPAD // NS
NPAD = 10240
RP = NPAD // NS
RSTG = 128

_mesh = plsc.VectorSubcoreMesh(core_axis_name="c", subcore_axis_name="s")


def _zero_fill_1d(ref, n):
    z = jnp.zeros((16,), jnp.float32)

    def body(i, _):
        ref[pl.ds(i * 16, 16)] = z
        return 0

    lax.fori_loop(0, n // 16, body, 0)


@functools.partial(
    pl.kernel,
    mesh=_mesh,
    out_type=[jax.ShapeDtypeStruct((HPAD,), jnp.float32),
              jax.ShapeDtypeStruct((HPAD,), jnp.float32)],
    scratch_types=[
        pltpu.VMEM_SHARED((HPAD,), jnp.float32),
        pltpu.VMEM((NCHUNK, C), jnp.int32),
        pltpu.VMEM((C,), jnp.float32),
        pltpu.VMEM((HP,), jnp.float32),
        pltpu.SemaphoreType.DMA,
        pltpu.SemaphoreType.DMA,
    ],
)
def _deg_kernel(dst_hbm, out0_hbm, out1_hbm, hist_sp, didx_v, ones_v, stage_v,
                sem0, sem1):
    cid = lax.axis_index("c")
    sid = lax.axis_index("s")
    wid = cid * NS + sid

    pltpu.sync_copy(dst_hbm.at[wid], didx_v)
    _zero_fill_1d(stage_v, HP)
    one = jnp.ones((16,), jnp.float32)

    def fill_ones(i, _):
        ones_v[pl.ds(i * 16, 16)] = one
        return 0

    lax.fori_loop(0, C // 16, fill_ones, 0)

    pltpu.sync_copy(stage_v, hist_sp.at[pl.ds(sid * HP, HP)])
    plsc.subcore_barrier()

    sem = (sem0, sem1)

    def add_ones(k, s):
        pltpu.async_copy(ones_v, hist_sp.at[didx_v.at[k]], sem[s], add=True)

    def add_wait(s):
        pltpu.make_async_copy(ones_v, hist_sp.at[didx_v.at[0]], sem[s]).wait()

    def chunk_pair(i, _):
        @pl.when(i >= 1)
        def _():
            add_wait(0)
        add_ones(2 * i, 0)

        @pl.when(i >= 1)
        def _():
            add_wait(1)
        add_ones(2 * i + 1, 1)
        return 0

    lax.fori_loop(0, (NCHUNK - 1) // 2, chunk_pair, 0)
    add_wait(0)
    add_ones(NCHUNK - 1, 0)
    add_wait(0)
    add_wait(1)
    plsc.subcore_barrier()

    @pl.when(cid == 0)
    def _():
        pltpu.sync_copy(hist_sp.at[pl.ds(sid * HP, HP)],
                        out0_hbm.at[pl.ds(sid * HP, HP)])

    @pl.when(cid == 1)
    def _():
        pltpu.sync_copy(hist_sp.at[pl.ds(sid * HP, HP)],
                        out1_hbm.at[pl.ds(sid * HP, HP)])


@functools.partial(
    pl.kernel,
    mesh=_mesh,
    out_type=[jax.ShapeDtypeStruct((NPAD, D), jnp.float32),
              jax.ShapeDtypeStruct((NPAD, D), jnp.float32)],
    scratch_types=[
        pltpu.VMEM_SHARED((NPAD, D), jnp.float32),
        pltpu.VMEM((C, D), jnp.float32),
        pltpu.VMEM((C, D), jnp.float32),
        pltpu.VMEM((EPP,), jnp.int32),
        pltpu.VMEM((NCHUNK, C), jnp.int32),
        pltpu.SemaphoreType.DMA,
        pltpu.SemaphoreType.DMA,
        pltpu.SemaphoreType.DMA,
        pltpu.SemaphoreType.DMA,
    ],
)
def _scatter_kernel(y_hbm, src_hbm, dst_hbm, out0_hbm, out1_hbm,
                    accum_sp, rows0_v, rows1_v, sidx_v, didx_v,
                    gsem0, gsem1, ssem0, ssem1):
    cid = lax.axis_index("c")
    sid = lax.axis_index("s")
    wid = cid * NS + sid

    pltpu.sync_copy(src_hbm.at[wid], sidx_v)
    pltpu.sync_copy(dst_hbm.at[wid], didx_v)

    z = jnp.zeros((16,), jnp.float32)

    def zrow(i, _):
        rows0_v[i // (D // 16), pl.ds((i % (D // 16)) * 16, 16)] = z
        return 0

    lax.fori_loop(0, C * (D // 16), zrow, 0)
    for j in range(RP // CP):
        pltpu.sync_copy(rows0_v.at[pl.ds(0, CP)],
                        accum_sp.at[pl.ds(sid * RP + j * CP, CP)])
    plsc.subcore_barrier()

    rows = (rows0_v, rows1_v)
    gsem = (gsem0, gsem1)
    ssem = (ssem0, ssem1)

    def gather(k, buf):
        pltpu.async_copy(y_hbm.at[sidx_v.at[pl.ds(k * C, C)]], rows[buf],
                         gsem[buf])

    def gather_wait(k, buf):
        pltpu.make_async_copy(y_hbm.at[sidx_v.at[pl.ds(k * C, C)]], rows[buf],
                              gsem[buf]).wait()

    def scatter(k, buf):
        pltpu.sync_copy(rows[buf], accum_sp.at[didx_v.at[k]], add=True)

    del ssem

    gather(0, 0)
    gather(1, 1)

    def pair(i, _):
        k0 = 2 * i
        k1 = k0 + 1
        gather_wait(k0, 0)
        scatter(k0, 0)
        gather(k0 + 2, 0)
        gather_wait(k1, 1)
        scatter(k1, 1)

        @pl.when(k1 + 2 <= NCHUNK - 1)
        def _():
            gather(k1 + 2, 1)
        return 0

    lax.fori_loop(0, (NCHUNK - 1) // 2, pair, 0)
    gather_wait(NCHUNK - 1, 0)
    scatter(NCHUNK - 1, 0)
    plsc.subcore_barrier()

    r = sid * RP

    @pl.when(cid == 0)
    def _():
        pltpu.sync_copy(accum_sp.at[pl.ds(r, RP)], out0_hbm.at[pl.ds(r, RP)])

    @pl.when(cid == 1)
    def _():
        pltpu.sync_copy(accum_sp.at[pl.ds(r, RP)], out1_hbm.at[pl.ds(r, RP)])


BN = 400
GRID = N // BN


def _layer1_body(x_ref, w_ref, p0_ref, p1_ref, y_ref, dinv_ref):
    deg = p0_ref[...] + p1_ref[...] + 1.0
    dinv = lax.rsqrt(deg)
    z = x_ref[...] * dinv
    y_ref[...] = jnp.dot(z, w_ref[...], preferred_element_type=jnp.float32)
    dinv_ref[...] = dinv


def _layer1_tc(x, W1, p0, p1):
    return pl.pallas_call(
        _layer1_body,
        grid=(GRID,),
        in_specs=[
            pl.BlockSpec((BN, D), lambda i: (i, 0)),
            pl.BlockSpec((D, D), lambda i: (0, 0)),
            pl.BlockSpec((BN, 1), lambda i: (i, 0)),
            pl.BlockSpec((BN, 1), lambda i: (i, 0)),
        ],
        out_specs=[
            pl.BlockSpec((BN, D), lambda i: (i, 0)),
            pl.BlockSpec((BN, 1), lambda i: (i, 0)),
        ],
        out_shape=[
            jax.ShapeDtypeStruct((N, D), jnp.float32),
            jax.ShapeDtypeStruct((N, 1), jnp.float32),
        ],
    )(x, W1, p0, p1)


def _mid_body(s0_ref, s1_ref, y1_ref, dinv_ref, w_ref, b_ref, y2_ref):
    dinv = dinv_ref[...]
    t = dinv * (s0_ref[...] + s1_ref[...] + y1_ref[...]) + b_ref[...]
    h = jnp.maximum(t, 0.0)
    y2_ref[...] = dinv * jnp.dot(h, w_ref[...],
                                 preferred_element_type=jnp.float32)


def _mid_tc(s0, s1, y1, dinv, W2, b1):
    return pl.pallas_call(
        _mid_body,
        grid=(GRID,),
        in_specs=[
            pl.BlockSpec((BN, D), lambda i: (i, 0)),
            pl.BlockSpec((BN, D), lambda i: (i, 0)),
            pl.BlockSpec((BN, D), lambda i: (i, 0)),
            pl.BlockSpec((BN, 1), lambda i: (i, 0)),
            pl.BlockSpec((D, D), lambda i: (0, 0)),
            pl.BlockSpec((1, D), lambda i: (0, 0)),
        ],
        out_specs=pl.BlockSpec((BN, D), lambda i: (i, 0)),
        out_shape=jax.ShapeDtypeStruct((N, D), jnp.float32),
    )(s0, s1, y1, dinv, W2, b1)


def _final_body(s0_ref, s1_ref, y2_ref, dinv_ref, b_ref, o_ref):
    o_ref[...] = (dinv_ref[...] * (s0_ref[...] + s1_ref[...] + y2_ref[...])
                  + b_ref[...])


def _final_tc(s0, s1, y2, dinv, b2):
    return pl.pallas_call(
        _final_body,
        grid=(GRID,),
        in_specs=[
            pl.BlockSpec((BN, D), lambda i: (i, 0)),
            pl.BlockSpec((BN, D), lambda i: (i, 0)),
            pl.BlockSpec((BN, D), lambda i: (i, 0)),
            pl.BlockSpec((BN, 1), lambda i: (i, 0)),
            pl.BlockSpec((1, D), lambda i: (0, 0)),
        ],
        out_specs=pl.BlockSpec((BN, D), lambda i: (i, 0)),
        out_shape=jax.ShapeDtypeStruct((N, D), jnp.float32),
    )(s0, s1, y2, dinv, b2)


def kernel(x, edge_index, W1, b1, W2, b2):
    src = edge_index[0]
    dst = edge_index[1]
    src2 = src.reshape(NW, EP)
    dst3 = dst.reshape(NW, NCHUNK, C)

    h0, h1 = _deg_kernel(dst3)
    y1, dinv = _layer1_tc(x, W1, h0.reshape(HPAD, 1), h1.reshape(HPAD, 1))

    s10, s11 = _scatter_kernel(y1, src2, dst3)
    y2 = _mid_tc(s10, s11, y1, dinv, W2, b1.reshape(1, D))

    s20, s21 = _scatter_kernel(y2, src2, dst3)
    out = _final_tc(s20, s21, y2, dinv, b2.reshape(1, D))
    return out


# speedup vs baseline: 1.5618x; 1.0008x over previous
import functools

import jax
import jax.numpy as jnp
from jax import lax
from jax.experimental import pallas as pl
from jax.experimental.pallas import tpu as pltpu
from jax.experimental.pallas import tpu_sc as plsc

N = 10000
E = 320000
D = 128

NC = 2
NS = 16
NW = NC * NS

EP = E // NW
C = 80
NCHUNK = EP // C
EPP = NCHUNK * C
CP = 80

HPAD = 10240
HP = HPAD // NS
NPAD = 10240
RP = NPAD // NS

_mesh = plsc.VectorSubcoreMesh(core_axis_name="c", subcore_axis_name="s")


def _zero_fill_1d(ref, n):
    z = jnp.zeros((16,), jnp.float32)

    def body(i, _):
        ref[pl.ds(i * 16, 16)] = z
        return 0

    lax.fori_loop(0, n // 16, body, 0)


@functools.partial(
    pl.kernel,
    mesh=_mesh,
    out_type=[jax.ShapeDtypeStruct((HPAD,), jnp.float32),
              jax.ShapeDtypeStruct((HPAD,), jnp.float32)],
    scratch_types=[
        pltpu.VMEM_SHARED((HPAD,), jnp.float32),
        pltpu.VMEM((NCHUNK, C), jnp.int32),
        pltpu.VMEM((C,), jnp.float32),
        pltpu.VMEM((HP,), jnp.float32),
        pltpu.SemaphoreType.DMA,
        pltpu.SemaphoreType.DMA,
    ],
)
def _deg_kernel(dst_hbm, out0_hbm, out1_hbm, hist_sp, didx_v, ones_v, stage_v,
                sem0, sem1):
    cid = lax.axis_index("c")
    sid = lax.axis_index("s")
    wid = cid * NS + sid

    pltpu.sync_copy(dst_hbm.at[wid], didx_v)
    _zero_fill_1d(stage_v, HP)
    one = jnp.ones((16,), jnp.float32)

    def fill_ones(i, _):
        ones_v[pl.ds(i * 16, 16)] = one
        return 0

    lax.fori_loop(0, C // 16, fill_ones, 0)

    pltpu.sync_copy(stage_v, hist_sp.at[pl.ds(sid * HP, HP)])
    plsc.subcore_barrier()

    sem = (sem0, sem1)

    def add_ones(k, s):
        pltpu.async_copy(ones_v, hist_sp.at[didx_v.at[k]], sem[s], add=True)

    def add_wait(s):
        pltpu.make_async_copy(ones_v, hist_sp.at[didx_v.at[0]], sem[s]).wait()

    def chunk_pair(i, _):
        @pl.when(i >= 1)
        def _():
            add_wait(0)
        add_ones(2 * i, 0)

        @pl.when(i >= 1)
        def _():
            add_wait(1)
        add_ones(2 * i + 1, 1)
        return 0

    lax.fori_loop(0, (NCHUNK - 1) // 2, chunk_pair, 0)
    add_wait(0)
    add_ones(NCHUNK - 1, 0)
    add_wait(0)
    add_wait(1)
    plsc.subcore_barrier()

    @pl.when(cid == 0)
    def _():
        pltpu.sync_copy(hist_sp.at[pl.ds(sid * HP, HP)],
                        out0_hbm.at[pl.ds(sid * HP, HP)])

    @pl.when(cid == 1)
    def _():
        pltpu.sync_copy(hist_sp.at[pl.ds(sid * HP, HP)],
                        out1_hbm.at[pl.ds(sid * HP, HP)])


@functools.partial(
    pl.kernel,
    mesh=_mesh,
    out_type=[jax.ShapeDtypeStruct((NPAD, D), jnp.float32),
              jax.ShapeDtypeStruct((NPAD, D), jnp.float32)],
    scratch_types=[
        pltpu.VMEM_SHARED((NPAD, D), jnp.float32),
        pltpu.VMEM((C, D), jnp.float32),
        pltpu.VMEM((C, D), jnp.float32),
        pltpu.VMEM((EPP,), jnp.int32),
        pltpu.VMEM((NCHUNK, C), jnp.int32),
        pltpu.SemaphoreType.DMA,
        pltpu.SemaphoreType.DMA,
        pltpu.SemaphoreType.DMA,
        pltpu.SemaphoreType.DMA,
    ],
)
def _scatter_kernel(y_hbm, src_hbm, dst_hbm, out0_hbm, out1_hbm,
                    accum_sp, rows0_v, rows1_v, sidx_v, didx_v,
                    gsem0, gsem1, ssem0, ssem1):
    cid = lax.axis_index("c")
    sid = lax.axis_index("s")
    wid = cid * NS + sid

    pltpu.sync_copy(src_hbm.at[wid], sidx_v)
    pltpu.sync_copy(dst_hbm.at[wid], didx_v)

    z = jnp.zeros((16,), jnp.float32)

    def zrow(i, _):
        rows0_v[i // (D // 16), pl.ds((i % (D // 16)) * 16, 16)] = z
        return 0

    lax.fori_loop(0, C * (D // 16), zrow, 0)
    for j in range(RP // CP):
        pltpu.sync_copy(rows0_v.at[pl.ds(0, CP)],
                        accum_sp.at[pl.ds(sid * RP + j * CP, CP)])
    plsc.subcore_barrier()

    rows = (rows0_v, rows1_v)
    gsem = (gsem0, gsem1)
    ssem = (ssem0, ssem1)

    def gather(k, buf):
        pltpu.async_copy(y_hbm.at[sidx_v.at[pl.ds(k * C, C)]], rows[buf],
                         gsem[buf])

    def gather_wait(k, buf):
        pltpu.make_async_copy(y_hbm.at[sidx_v.at[pl.ds(k * C, C)]], rows[buf],
                              gsem[buf]).wait()

    def scatter(k, buf):
        pltpu.sync_copy(rows[buf], accum_sp.at[didx_v.at[k]], add=True)

    del ssem

    gather(0, 0)
    gather(1, 1)

    def pair(i, _):
        k0 = 2 * i
        k1 = k0 + 1
        gather_wait(k0, 0)
        scatter(k0, 0)
        gather(k0 + 2, 0)
        gather_wait(k1, 1)
        scatter(k1, 1)

        @pl.when(k1 + 2 <= NCHUNK - 1)
        def _():
            gather(k1 + 2, 1)
        return 0

    lax.fori_loop(0, (NCHUNK - 1) // 2, pair, 0)
    gather_wait(NCHUNK - 1, 0)
    scatter(NCHUNK - 1, 0)
    plsc.subcore_barrier()

    r = sid * RP

    @pl.when(cid == 0)
    def _():
        pltpu.sync_copy(accum_sp.at[pl.ds(r, RP)], out0_hbm.at[pl.ds(r, RP)])

    @pl.when(cid == 1)
    def _():
        pltpu.sync_copy(accum_sp.at[pl.ds(r, RP)], out1_hbm.at[pl.ds(r, RP)])


BN = 400
GRID = N // BN


def _layer1_body(x_ref, w_ref, p0_ref, p1_ref, y_ref, dinv_ref):
    deg = p0_ref[...] + p1_ref[...] + 1.0
    dinv = lax.rsqrt(deg)
    z = x_ref[...] * dinv
    y_ref[...] = jnp.dot(z, w_ref[...], preferred_element_type=jnp.float32)
    dinv_ref[...] = dinv


def _layer1_tc(x, W1, p0, p1):
    return pl.pallas_call(
        _layer1_body,
        grid=(GRID,),
        in_specs=[
            pl.BlockSpec((BN, D), lambda i: (i, 0)),
            pl.BlockSpec((D, D), lambda i: (0, 0)),
            pl.BlockSpec((BN, 1), lambda i: (i, 0)),
            pl.BlockSpec((BN, 1), lambda i: (i, 0)),
        ],
        out_specs=[
            pl.BlockSpec((BN, D), lambda i: (i, 0)),
            pl.BlockSpec((BN, 1), lambda i: (i, 0)),
        ],
        out_shape=[
            jax.ShapeDtypeStruct((N, D), jnp.float32),
            jax.ShapeDtypeStruct((N, 1), jnp.float32),
        ],
    )(x, W1, p0, p1)


def _mid_body(s0_ref, s1_ref, y1_ref, dinv_ref, w_ref, b_ref, y2_ref):
    dinv = dinv_ref[...]
    t = dinv * (s0_ref[...] + s1_ref[...] + y1_ref[...]) + b_ref[...]
    h = jnp.maximum(t, 0.0)
    y2_ref[...] = dinv * jnp.dot(h, w_ref[...],
                                 preferred_element_type=jnp.float32)


def _mid_tc(s0, s1, y1, dinv, W2, b1):
    return pl.pallas_call(
        _mid_body,
        grid=(GRID,),
        in_specs=[
            pl.BlockSpec((BN, D), lambda i: (i, 0)),
            pl.BlockSpec((BN, D), lambda i: (i, 0)),
            pl.BlockSpec((BN, D), lambda i: (i, 0)),
            pl.BlockSpec((BN, 1), lambda i: (i, 0)),
            pl.BlockSpec((D, D), lambda i: (0, 0)),
            pl.BlockSpec((1, D), lambda i: (0, 0)),
        ],
        out_specs=pl.BlockSpec((BN, D), lambda i: (i, 0)),
        out_shape=jax.ShapeDtypeStruct((N, D), jnp.float32),
    )(s0, s1, y1, dinv, W2, b1)


def _final_body(s0_ref, s1_ref, y2_ref, dinv_ref, b_ref, o_ref):
    o_ref[...] = (dinv_ref[...] * (s0_ref[...] + s1_ref[...] + y2_ref[...])
                  + b_ref[...])


def _final_tc(s0, s1, y2, dinv, b2):
    return pl.pallas_call(
        _final_body,
        grid=(GRID,),
        in_specs=[
            pl.BlockSpec((BN, D), lambda i: (i, 0)),
            pl.BlockSpec((BN, D), lambda i: (i, 0)),
            pl.BlockSpec((BN, D), lambda i: (i, 0)),
            pl.BlockSpec((BN, 1), lambda i: (i, 0)),
            pl.BlockSpec((1, D), lambda i: (0, 0)),
        ],
        out_specs=pl.BlockSpec((BN, D), lambda i: (i, 0)),
        out_shape=jax.ShapeDtypeStruct((N, D), jnp.float32),
    )(s0, s1, y2, dinv, b2)


def kernel(x, edge_index, W1, b1, W2, b2):
    src = edge_index[0]
    dst = edge_index[1]
    src2 = src.reshape(NW, EP)
    dst3 = dst.reshape(NW, NCHUNK, C)

    h0, h1 = _deg_kernel(dst3)
    y1, dinv = _layer1_tc(x, W1, h0.reshape(HPAD, 1), h1.reshape(HPAD, 1))

    s10, s11 = _scatter_kernel(y1, src2, dst3)
    y2 = _mid_tc(s10, s11, y1, dinv, W2, b1.reshape(1, D))

    s20, s21 = _scatter_kernel(y2, src2, dst3)
    out = _final_tc(s20, s21, y2, dinv, b2.reshape(1, D))
    return out
